# baseline (speedup 1.0000x reference)
import jax
import jax.numpy as jnp
from jax.experimental import pallas as pl
from jax.experimental.pallas import tpu as pltpu

_BM = 512
_NBUF = 4


def _router_manual(hs_ref, w_ref, o_ref, xbuf, obuf, in_sems, out_sems):
    nsteps = hs_ref.shape[0] // _BM

    def in_copy(blk, slot):
        return pltpu.make_async_copy(
            hs_ref.at[pl.ds(blk * _BM, _BM), :],
            xbuf.at[slot],
            in_sems.at[slot],
        )

    def out_copy(blk, slot):
        return pltpu.make_async_copy(
            obuf.at[slot],
            o_ref.at[pl.ds(blk * _BM, _BM), :],
            out_sems.at[slot],
        )

    for slot in range(_NBUF):
        in_copy(slot, slot).start()

    for blk in range(nsteps):
        slot = blk % _NBUF
        in_copy(blk, slot).wait()
        if blk >= _NBUF:
            out_copy(blk - _NBUF, slot).wait()
        obuf[slot] = jax.lax.dot_general(
            xbuf[slot], w_ref[...],
            dimension_numbers=(((1,), (1,)), ((), ())),
            preferred_element_type=jnp.float32,
            precision=jax.lax.Precision.DEFAULT,
        )
        out_copy(blk, slot).start()
        if blk + _NBUF < nsteps:
            in_copy(blk + _NBUF, slot).start()

    for blk in range(nsteps - _NBUF, nsteps):
        out_copy(blk, blk % _NBUF).wait()


def kernel(hidden_states, W):
    hs = hidden_states.reshape(-1, hidden_states.shape[-1])
    m, k = hs.shape
    n = W.shape[0]
    return pl.pallas_call(
        _router_manual,
        in_specs=[
            pl.BlockSpec(memory_space=pltpu.MemorySpace.HBM),
            pl.BlockSpec(memory_space=pltpu.MemorySpace.VMEM),
        ],
        out_specs=pl.BlockSpec(memory_space=pltpu.MemorySpace.HBM),
        out_shape=jax.ShapeDtypeStruct((m, n), jnp.float32),
        scratch_shapes=[
            pltpu.VMEM((_NBUF, _BM, k), jnp.float32),
            pltpu.VMEM((_NBUF, _BM, n), jnp.float32),
            pltpu.SemaphoreType.DMA((_NBUF,)),
            pltpu.SemaphoreType.DMA((_NBUF,)),
        ],
        compiler_params=pltpu.CompilerParams(
            vmem_limit_bytes=64 * 1024 * 1024,
        ),
    )(hs, W)


# speedup vs baseline: 1.0363x; 1.0363x over previous
import jax
import jax.numpy as jnp
from jax.experimental import pallas as pl
from jax.experimental.pallas import tpu as pltpu

_BM = 512


def _router_block(x_ref, w_ref, o_ref):
    i = pl.program_id(0)
    o_ref[pl.ds(i * _BM, _BM), :] = jax.lax.dot_general(
        x_ref[...], w_ref[...],
        dimension_numbers=(((1,), (1,)), ((), ())),
        preferred_element_type=jnp.float32,
        precision=jax.lax.Precision.DEFAULT,
    )


def kernel(hidden_states, W):
    hs = hidden_states.reshape(-1, hidden_states.shape[-1])
    m, k = hs.shape
    n = W.shape[0]
    return pl.pallas_call(
        _router_block,
        grid=(m // _BM,),
        in_specs=[
            pl.BlockSpec((_BM, k), lambda i: (i, 0)),
            pl.BlockSpec((n, k), lambda i: (0, 0)),
        ],
        out_specs=pl.BlockSpec((m, n), lambda i: (0, 0)),
        out_shape=jax.ShapeDtypeStruct((m, n), jnp.float32),
        compiler_params=pltpu.CompilerParams(
            dimension_semantics=("arbitrary",),
        ),
    )(hs, W)
